# baseline (speedup 1.0000x reference)
import functools

import jax
import jax.numpy as jnp
from jax import lax
from jax.experimental import pallas as pl
from jax.experimental.pallas import tpu as pltpu
from jax.experimental.pallas import tpu_sc as plsc

VOCAB = 1000000
EMB = 64
BATCH = 4096
HIST = 200

N = BATCH * HIST
NC, NS = 1, 16
NW = NC * NS
PER_W = N // NW
ROW = 128
ROWS_PER_W = PER_W // ROW
CHUNK_ROWS = 4
CHUNK = CHUNK_ROWS * ROW
N_ITERS = ROWS_PER_W // CHUNK_ROWS
NBUF = 2


@functools.partial(
    pl.kernel,
    out_type=jax.ShapeDtypeStruct((N, EMB), jnp.float32),
    mesh=plsc.VectorSubcoreMesh(
        core_axis_name="c", subcore_axis_name="s", num_cores=NC
    ),
    scratch_types=[
        pltpu.VMEM((NBUF, CHUNK_ROWS, ROW), jnp.int32),
        pltpu.VMEM((NBUF, CHUNK, EMB), jnp.float32),
        pltpu.SemaphoreType.DMA((NBUF,)),
        pltpu.SemaphoreType.DMA((NBUF,)),
        pltpu.SemaphoreType.DMA((NBUF,)),
    ],
    compiler_params=pltpu.CompilerParams(use_tc_tiling_on_sc=False),
)
def _sc_embed(x_hbm, table_hbm, out_hbm, idx_v, rows_v, sem_i, sem_g, sem_o):
    wid = lax.axis_index("s") * NC + lax.axis_index("c")
    base_row = wid * ROWS_PER_W

    def idx_dma(g, p):
        row_off = base_row + g * CHUNK_ROWS
        return pltpu.make_async_copy(
            x_hbm.at[pl.ds(row_off, CHUNK_ROWS)], idx_v.at[p], sem_i.at[p]
        )

    def gather_dma(p, j):
        return pltpu.make_async_copy(
            table_hbm.at[idx_v.at[p, j]],
            rows_v.at[p, pl.ds(j * ROW, ROW)],
            sem_g.at[p],
        )

    def out_dma(g, p):
        off = (base_row + g * CHUNK_ROWS) * ROW
        return pltpu.make_async_copy(
            rows_v.at[p], out_hbm.at[pl.ds(off, CHUNK)], sem_o.at[p]
        )

    def step(g, p, q, first):
        for j in range(CHUNK_ROWS):
            gather_dma(p, j).wait()
        out_dma(g, p).start()
        idx_dma(jnp.minimum(g + 2, N_ITERS - 1), p).start()
        idx_dma(g + 1, q).wait()
        if not first:
            out_dma(g, q).wait()
        for j in range(CHUNK_ROWS):
            gather_dma(q, j).start()

    idx_dma(0, 0).start()
    idx_dma(1, 1).start()
    idx_dma(0, 0).wait()
    for j in range(CHUNK_ROWS):
        gather_dma(0, j).start()

    step(0, 0, 1, True)

    def pair(k, carry):
        g = 1 + 2 * k
        step(g, 1, 0, False)
        step(g + 1, 0, 1, False)
        return carry

    lax.fori_loop(0, (N_ITERS - 2) // 2, pair, 0)

    for j in range(CHUNK_ROWS):
        gather_dma(1, j).wait()
    out_dma(N_ITERS - 1, 1).start()
    idx_dma(N_ITERS - 1, 0).wait()
    out_dma(N_ITERS - 2, 0).wait()
    out_dma(N_ITERS - 1, 1).wait()


def _mask_body(x_ref, mask_ref):
    mask_ref[...] = (x_ref[...] != 0).astype(jnp.float32)


def _mask_tc(x):
    return pl.pallas_call(
        _mask_body,
        out_shape=jax.ShapeDtypeStruct((BATCH, HIST), jnp.float32),
        grid=(8,),
        in_specs=[pl.BlockSpec((BATCH // 8, HIST), lambda i: (i, 0))],
        out_specs=pl.BlockSpec((BATCH // 8, HIST), lambda i: (i, 0)),
    )(x)


def kernel(x, table):
    x2d = x.reshape(N // ROW, ROW)
    out = _sc_embed(x2d, table)
    mask = _mask_tc(x)
    return out.reshape(BATCH, HIST, EMB), mask


# speedup vs baseline: 1.0582x; 1.0582x over previous
import functools

import jax
import jax.numpy as jnp
from jax import lax
from jax.experimental import pallas as pl
from jax.experimental.pallas import tpu as pltpu
from jax.experimental.pallas import tpu_sc as plsc

VOCAB = 1000000
EMB = 64
BATCH = 4096
HIST = 200

NC, NS = 2, 16
NW = NC * NS
B_PER_W = BATCH // NW
BPC = 4
N_ITERS = B_PER_W // BPC
SPLITS = (0, 128)
SIZES = (128, 72)
NBUF = 2


@functools.partial(
    pl.kernel,
    out_type=jax.ShapeDtypeStruct((BATCH, HIST, EMB), jnp.float32),
    mesh=plsc.VectorSubcoreMesh(
        core_axis_name="c", subcore_axis_name="s", num_cores=NC
    ),
    scratch_types=[
        pltpu.VMEM((NBUF, BPC, HIST), jnp.int32),
        pltpu.VMEM((NBUF, BPC, HIST, EMB), jnp.float32),
        pltpu.SemaphoreType.DMA((NBUF,)),
        pltpu.SemaphoreType.DMA((NBUF,)),
        pltpu.SemaphoreType.DMA((NBUF,)),
    ],
    compiler_params=pltpu.CompilerParams(use_tc_tiling_on_sc=False),
)
def _sc_embed(x_hbm, table_hbm, out_hbm, idx_v, rows_v, sem_i, sem_g, sem_o):
    wid = lax.axis_index("s") * NC + lax.axis_index("c")
    base_b = wid * B_PER_W

    def idx_dma(g, p):
        b0 = base_b + g * BPC
        return pltpu.make_async_copy(
            x_hbm.at[pl.ds(b0, BPC)], idx_v.at[p], sem_i.at[p]
        )

    def gather_dma(p, j, k):
        return pltpu.make_async_copy(
            table_hbm.at[idx_v.at[p, j, pl.ds(SPLITS[k], SIZES[k])]],
            rows_v.at[p, j, pl.ds(SPLITS[k], SIZES[k])],
            sem_g.at[p],
        )

    def gathers(fn_name, p):
        for j in range(BPC):
            for k in range(len(SIZES)):
                getattr(gather_dma(p, j, k), fn_name)()

    def out_dma(g, p):
        b0 = base_b + g * BPC
        return pltpu.make_async_copy(
            rows_v.at[p], out_hbm.at[pl.ds(b0, BPC)], sem_o.at[p]
        )

    def step(g, p, q, first):
        gathers("wait", p)
        out_dma(g, p).start()
        idx_dma(jnp.minimum(g + 2, N_ITERS - 1), p).start()
        idx_dma(g + 1, q).wait()
        if not first:
            out_dma(g, q).wait()
        gathers("start", q)

    idx_dma(0, 0).start()
    idx_dma(1, 1).start()
    idx_dma(0, 0).wait()
    gathers("start", 0)

    step(0, 0, 1, True)

    def pair(k, carry):
        g = 1 + 2 * k
        step(g, 1, 0, False)
        step(g + 1, 0, 1, False)
        return carry

    lax.fori_loop(0, (N_ITERS - 2) // 2, pair, 0)

    gathers("wait", 1)
    out_dma(N_ITERS - 1, 1).start()
    idx_dma(N_ITERS - 1, 0).wait()
    out_dma(N_ITERS - 2, 0).wait()
    out_dma(N_ITERS - 1, 1).wait()


def _mask_body(x_ref, mask_ref):
    mask_ref[...] = (x_ref[...] != 0).astype(jnp.float32)


def _mask_tc(x):
    return pl.pallas_call(
        _mask_body,
        out_shape=jax.ShapeDtypeStruct((BATCH, HIST), jnp.float32),
        grid=(8,),
        in_specs=[pl.BlockSpec((BATCH // 8, HIST), lambda i: (i, 0))],
        out_specs=pl.BlockSpec((BATCH // 8, HIST), lambda i: (i, 0)),
    )(x)


def kernel(x, table):
    out = _sc_embed(x, table)
    mask = _mask_tc(x)
    return out, mask


# speedup vs baseline: 1.2851x; 1.2144x over previous
import functools

import jax
import jax.numpy as jnp
from jax import lax
from jax.experimental import pallas as pl
from jax.experimental.pallas import tpu as pltpu
from jax.experimental.pallas import tpu_sc as plsc

VOCAB = 1000000
EMB = 64
PAD = 128
BATCH = 4096
HIST = 200

NC, NS = 2, 16
NW = NC * NS
B_PER_W = BATCH // NW
BPC = 2
N_ITERS = B_PER_W // BPC
SPLITS = (0, 128)
SIZES = (128, 72)
NBUF = 2


@functools.partial(
    pl.kernel,
    out_type=jax.ShapeDtypeStruct((BATCH, HIST, PAD), jnp.float32),
    mesh=plsc.VectorSubcoreMesh(
        core_axis_name="c", subcore_axis_name="s", num_cores=NC
    ),
    scratch_types=[
        pltpu.VMEM((NBUF, BPC, HIST), jnp.int32),
        pltpu.VMEM((NBUF, BPC, HIST, PAD), jnp.float32),
        pltpu.SemaphoreType.DMA((NBUF,)),
        pltpu.SemaphoreType.DMA((NBUF,)),
        pltpu.SemaphoreType.DMA((NBUF,)),
    ],
    compiler_params=pltpu.CompilerParams(use_tc_tiling_on_sc=True),
)
def _sc_embed(x_hbm, table_hbm, out_hbm, idx_v, rows_v, sem_i, sem_g, sem_o):
    wid = lax.axis_index("s") * NC + lax.axis_index("c")
    base_b = wid * B_PER_W

    def idx_dma(g, p):
        b0 = base_b + g * BPC
        return pltpu.make_async_copy(
            x_hbm.at[pl.ds(b0, BPC)], idx_v.at[p], sem_i.at[p]
        )

    def gather_dma(p, j, k):
        return pltpu.make_async_copy(
            table_hbm.at[idx_v.at[p, j, pl.ds(SPLITS[k], SIZES[k])]],
            rows_v.at[p, j, pl.ds(SPLITS[k], SIZES[k])],
            sem_g.at[p],
        )

    def gathers(fn_name, p):
        for j in range(BPC):
            for k in range(len(SIZES)):
                getattr(gather_dma(p, j, k), fn_name)()

    def out_dma(g, p):
        b0 = base_b + g * BPC
        return pltpu.make_async_copy(
            rows_v.at[p], out_hbm.at[pl.ds(b0, BPC)], sem_o.at[p]
        )

    def step(g, p, q, first):
        gathers("wait", p)
        out_dma(g, p).start()
        idx_dma(jnp.minimum(g + 2, N_ITERS - 1), p).start()
        idx_dma(g + 1, q).wait()
        if not first:
            out_dma(g, q).wait()
        gathers("start", q)

    idx_dma(0, 0).start()
    idx_dma(1, 1).start()
    idx_dma(0, 0).wait()
    gathers("start", 0)

    step(0, 0, 1, True)

    def pair(k, carry):
        g = 1 + 2 * k
        step(g, 1, 0, False)
        step(g + 1, 0, 1, False)
        return carry

    lax.fori_loop(0, (N_ITERS - 2) // 2, pair, 0)

    gathers("wait", 1)
    out_dma(N_ITERS - 1, 1).start()
    idx_dma(N_ITERS - 1, 0).wait()
    out_dma(N_ITERS - 2, 0).wait()
    out_dma(N_ITERS - 1, 1).wait()


def _mask_body(x_ref, mask_ref):
    mask_ref[...] = (x_ref[...] != 0).astype(jnp.float32)


def _mask_tc(x):
    return pl.pallas_call(
        _mask_body,
        out_shape=jax.ShapeDtypeStruct((BATCH, HIST), jnp.float32),
        grid=(8,),
        in_specs=[pl.BlockSpec((BATCH // 8, HIST), lambda i: (i, 0))],
        out_specs=pl.BlockSpec((BATCH // 8, HIST), lambda i: (i, 0)),
    )(x)


def kernel(x, table):
    table_pad = jnp.pad(table, ((0, 0), (0, PAD - EMB)))
    out_pad = _sc_embed(x, table_pad)
    out = out_pad[:, :, :EMB]
    mask = _mask_tc(x)
    return out, mask


# speedup vs baseline: 1.5665x; 1.2190x over previous
import functools

import jax
import jax.numpy as jnp
from jax import lax
from jax.experimental import pallas as pl
from jax.experimental.pallas import tpu as pltpu
from jax.experimental.pallas import tpu_sc as plsc

VOCAB = 1000000
EMB = 64
PAD = 128
BATCH = 4096
HIST = 200

NC, NS = 2, 16
NW = NC * NS
B_PER_W = BATCH // NW
BPC = 2
N_ITERS = B_PER_W // BPC
SPLITS = (0, 128)
SIZES = (128, 72)
NBUF = 2


@functools.partial(
    pl.kernel,
    out_type=jax.ShapeDtypeStruct((BATCH, HIST, PAD), jnp.float32),
    mesh=plsc.VectorSubcoreMesh(
        core_axis_name="c", subcore_axis_name="s", num_cores=NC
    ),
    scratch_types=[
        pltpu.VMEM((NBUF, BPC, HIST), jnp.int32),
        pltpu.VMEM((NBUF, BPC, HIST, PAD), jnp.float32),
        pltpu.SemaphoreType.DMA((NBUF,)),
        pltpu.SemaphoreType.DMA((NBUF,)),
        pltpu.SemaphoreType.DMA((NBUF,)),
    ],
    compiler_params=pltpu.CompilerParams(use_tc_tiling_on_sc=True),
)
def _sc_embed(x_hbm, table_hbm, out_hbm, idx_v, rows_v, sem_i, sem_g, sem_o):
    wid = lax.axis_index("s") * NC + lax.axis_index("c")
    base_b = wid * B_PER_W

    def idx_dma(g, p):
        b0 = base_b + g * BPC
        return pltpu.make_async_copy(
            x_hbm.at[pl.ds(b0, BPC)], idx_v.at[p], sem_i.at[p]
        )

    def gather_dma(p, j, k):
        return pltpu.make_async_copy(
            table_hbm.at[idx_v.at[p, j, pl.ds(SPLITS[k], SIZES[k])]],
            rows_v.at[p, j, pl.ds(SPLITS[k], SIZES[k])],
            sem_g.at[p],
        )

    def gathers(fn_name, p):
        for j in range(BPC):
            for k in range(len(SIZES)):
                getattr(gather_dma(p, j, k), fn_name)()

    def out_dma(g, p):
        b0 = base_b + g * BPC
        return pltpu.make_async_copy(
            rows_v.at[p], out_hbm.at[pl.ds(b0, BPC)], sem_o.at[p]
        )

    def step(g, p, q, first):
        gathers("wait", p)
        out_dma(g, p).start()
        idx_dma(jnp.minimum(g + 2, N_ITERS - 1), p).start()
        idx_dma(g + 1, q).wait()
        if not first:
            out_dma(g, q).wait()
        gathers("start", q)

    idx_dma(0, 0).start()
    idx_dma(1, 1).start()
    idx_dma(0, 0).wait()
    gathers("start", 0)

    step(0, 0, 1, True)

    def pair(k, carry):
        g = 1 + 2 * k
        step(g, 1, 0, False)
        step(g + 1, 0, 1, False)
        return carry

    lax.fori_loop(0, (N_ITERS - 2) // 2, pair, 0)

    gathers("wait", 1)
    out_dma(N_ITERS - 1, 1).start()
    idx_dma(N_ITERS - 1, 0).wait()
    out_dma(N_ITERS - 2, 0).wait()
    out_dma(N_ITERS - 1, 1).wait()


TBLK = 3840


def _tpose_body(tt_ref, out_ref):
    t = tt_ref[...].T
    out_ref[...] = jnp.concatenate([t, jnp.zeros_like(t)], axis=1)


def _table_pad_tc(table):
    return pl.pallas_call(
        _tpose_body,
        out_shape=jax.ShapeDtypeStruct((VOCAB, PAD), jnp.float32),
        grid=(pl.cdiv(VOCAB, TBLK),),
        in_specs=[pl.BlockSpec((EMB, TBLK), lambda i: (0, i))],
        out_specs=pl.BlockSpec((TBLK, PAD), lambda i: (i, 0)),
    )(table.T)


def _mask_body(x_ref, mask_ref):
    mask_ref[...] = (x_ref[...] != 0).astype(jnp.float32)


def _mask_tc(x):
    return pl.pallas_call(
        _mask_body,
        out_shape=jax.ShapeDtypeStruct((BATCH, HIST), jnp.float32),
        grid=(8,),
        in_specs=[pl.BlockSpec((BATCH // 8, HIST), lambda i: (i, 0))],
        out_specs=pl.BlockSpec((BATCH // 8, HIST), lambda i: (i, 0)),
    )(x)


def kernel(x, table):
    table_pad = _table_pad_tc(table)
    out_pad = _sc_embed(x, table_pad)
    out = out_pad[:, :, :EMB]
    mask = _mask_tc(x)
    return out, mask


# speedup vs baseline: 1.7170x; 1.0961x over previous
import functools

import jax
import jax.numpy as jnp
from jax import lax
from jax.experimental import pallas as pl
from jax.experimental.pallas import tpu as pltpu
from jax.experimental.pallas import tpu_sc as plsc

VOCAB = 1000000
EMB = 64
PAD = 128
BATCH = 4096
HIST = 200

NC, NS = 2, 16
NW = NC * NS
B_PER_W = BATCH // NW
BPC = 2
N_ITERS = B_PER_W // BPC
SPLITS = (0, 128)
SIZES = (128, 72)
NBUF = 2


@functools.partial(
    pl.kernel,
    out_type=jax.ShapeDtypeStruct((BATCH, HIST, PAD), jnp.float32),
    mesh=plsc.VectorSubcoreMesh(
        core_axis_name="c", subcore_axis_name="s", num_cores=NC
    ),
    scratch_types=[
        pltpu.VMEM((NBUF, BPC, HIST), jnp.int32),
        pltpu.VMEM((NBUF, BPC, HIST, PAD), jnp.float32),
        pltpu.SemaphoreType.DMA((NBUF,)),
        pltpu.SemaphoreType.DMA((NBUF,)),
        pltpu.SemaphoreType.DMA((NBUF,)),
    ],
    compiler_params=pltpu.CompilerParams(use_tc_tiling_on_sc=True),
)
def _sc_embed(x_hbm, table_hbm, out_hbm, idx_v, rows_v, sem_i, sem_g, sem_o):
    wid = lax.axis_index("s") * NC + lax.axis_index("c")
    base_b = wid * B_PER_W

    def idx_dma(g, p):
        b0 = base_b + g * BPC
        return pltpu.make_async_copy(
            x_hbm.at[pl.ds(b0, BPC)], idx_v.at[p], sem_i.at[p]
        )

    def gather_dma(p, j, k):
        return pltpu.make_async_copy(
            table_hbm.at[idx_v.at[p, j, pl.ds(SPLITS[k], SIZES[k])]],
            rows_v.at[p, j, pl.ds(SPLITS[k], SIZES[k])],
            sem_g.at[p],
        )

    def gathers(fn_name, p):
        for j in range(BPC):
            for k in range(len(SIZES)):
                getattr(gather_dma(p, j, k), fn_name)()

    def out_dma(g, p):
        b0 = base_b + g * BPC
        return pltpu.make_async_copy(
            rows_v.at[p], out_hbm.at[pl.ds(b0, BPC)], sem_o.at[p]
        )

    def step(g, p, q, first):
        gathers("wait", p)
        out_dma(g, p).start()
        idx_dma(jnp.minimum(g + 2, N_ITERS - 1), p).start()
        idx_dma(g + 1, q).wait()
        if not first:
            out_dma(g, q).wait()
        gathers("start", q)

    idx_dma(0, 0).start()
    idx_dma(1, 1).start()
    idx_dma(0, 0).wait()
    gathers("start", 0)

    step(0, 0, 1, True)

    def pair(k, carry):
        g = 1 + 2 * k
        step(g, 1, 0, False)
        step(g + 1, 0, 1, False)
        return carry

    lax.fori_loop(0, (N_ITERS - 2) // 2, pair, 0)

    gathers("wait", 1)
    out_dma(N_ITERS - 1, 1).start()
    idx_dma(N_ITERS - 1, 0).wait()
    out_dma(N_ITERS - 2, 0).wait()
    out_dma(N_ITERS - 1, 1).wait()


TBLK = 7680


def _tpose_body(tt_ref, out_ref):
    t = tt_ref[...].T
    out_ref[...] = jnp.concatenate([t, jnp.zeros_like(t)], axis=1)


def _table_pad_tc(table):
    return pl.pallas_call(
        _tpose_body,
        out_shape=jax.ShapeDtypeStruct((VOCAB, PAD), jnp.float32),
        grid=(pl.cdiv(VOCAB, TBLK),),
        in_specs=[pl.BlockSpec((EMB, TBLK), lambda i: (0, i))],
        out_specs=pl.BlockSpec((TBLK, PAD), lambda i: (i, 0)),
    )(table.T)


def _mask_body(x_ref, mask_ref):
    mask_ref[...] = (x_ref[...] != 0).astype(jnp.float32)


def _mask_tc(x):
    return pl.pallas_call(
        _mask_body,
        out_shape=jax.ShapeDtypeStruct((BATCH, HIST), jnp.float32),
        grid=(8,),
        in_specs=[pl.BlockSpec((BATCH // 8, HIST), lambda i: (i, 0))],
        out_specs=pl.BlockSpec((BATCH // 8, HIST), lambda i: (i, 0)),
    )(x)


def kernel(x, table):
    table_pad = _table_pad_tc(table)
    out_pad = _sc_embed(x, table_pad)
    out = out_pad[:, :, :EMB]
    mask = _mask_tc(x)
    return out, mask


# speedup vs baseline: 1.7771x; 1.0350x over previous
import functools

import jax
import jax.numpy as jnp
from jax import lax
from jax.experimental import pallas as pl
from jax.experimental.pallas import tpu as pltpu
from jax.experimental.pallas import tpu_sc as plsc

VOCAB = 1000000
EMB = 64
PAD = 128
BATCH = 4096
HIST = 200

NC, NS = 2, 16
NW = NC * NS
B_PER_W = BATCH // NW
BPC = 2
N_ITERS = B_PER_W // BPC
SPLITS = (0, 128)
SIZES = (128, 72)
NBUF = 2


@functools.partial(
    pl.kernel,
    out_type=jax.ShapeDtypeStruct((BATCH, HIST, PAD), jnp.float32),
    mesh=plsc.VectorSubcoreMesh(
        core_axis_name="c", subcore_axis_name="s", num_cores=NC
    ),
    scratch_types=[
        pltpu.VMEM((NBUF, BPC, HIST), jnp.int32),
        pltpu.VMEM((NBUF, BPC, HIST, PAD), jnp.float32),
        pltpu.SemaphoreType.DMA((NBUF,)),
        pltpu.SemaphoreType.DMA((NBUF,)),
        pltpu.SemaphoreType.DMA((NBUF,)),
    ],
    compiler_params=pltpu.CompilerParams(use_tc_tiling_on_sc=True),
)
def _sc_embed(x_hbm, table_hbm, out_hbm, idx_v, rows_v, sem_i, sem_g, sem_o):
    wid = lax.axis_index("s") * NC + lax.axis_index("c")
    base_b = wid * B_PER_W

    def idx_dma(g, p):
        b0 = base_b + g * BPC
        return pltpu.make_async_copy(
            x_hbm.at[pl.ds(b0, BPC)], idx_v.at[p], sem_i.at[p]
        )

    def gather_dma(p, j, k):
        return pltpu.make_async_copy(
            table_hbm.at[idx_v.at[p, j, pl.ds(SPLITS[k], SIZES[k])]],
            rows_v.at[p, j, pl.ds(SPLITS[k], SIZES[k])],
            sem_g.at[p],
        )

    def gathers(fn_name, p):
        for j in range(BPC):
            for k in range(len(SIZES)):
                getattr(gather_dma(p, j, k), fn_name)()

    def out_dma(g, p):
        b0 = base_b + g * BPC
        return pltpu.make_async_copy(
            rows_v.at[p], out_hbm.at[pl.ds(b0, BPC)], sem_o.at[p]
        )

    def step(g, p, q, first):
        gathers("wait", p)
        out_dma(g, p).start()
        idx_dma(jnp.minimum(g + 2, N_ITERS - 1), p).start()
        idx_dma(g + 1, q).wait()
        if not first:
            out_dma(g, q).wait()
        gathers("start", q)

    idx_dma(0, 0).start()
    idx_dma(1, 1).start()
    idx_dma(0, 0).wait()
    gathers("start", 0)

    step(0, 0, 1, True)

    def pair(k, carry):
        g = 1 + 2 * k
        step(g, 1, 0, False)
        step(g + 1, 0, 1, False)
        return carry

    lax.fori_loop(0, (N_ITERS - 2) // 2, pair, 0)

    gathers("wait", 1)
    out_dma(N_ITERS - 1, 1).start()
    idx_dma(N_ITERS - 1, 0).wait()
    out_dma(N_ITERS - 2, 0).wait()
    out_dma(N_ITERS - 1, 1).wait()


TBLK = 15360


def _tpose_body(tt_ref, out_ref):
    t = tt_ref[...].T
    out_ref[...] = jnp.concatenate([t, jnp.zeros_like(t)], axis=1)


def _table_pad_tc(table):
    return pl.pallas_call(
        _tpose_body,
        out_shape=jax.ShapeDtypeStruct((VOCAB, PAD), jnp.float32),
        grid=(pl.cdiv(VOCAB, TBLK),),
        in_specs=[pl.BlockSpec((EMB, TBLK), lambda i: (0, i))],
        out_specs=pl.BlockSpec((TBLK, PAD), lambda i: (i, 0)),
    )(table.T)


def _mask_body(x_ref, mask_ref):
    mask_ref[...] = (x_ref[...] != 0).astype(jnp.float32)


def _mask_tc(x):
    return pl.pallas_call(
        _mask_body,
        out_shape=jax.ShapeDtypeStruct((BATCH, HIST), jnp.float32),
        grid=(8,),
        in_specs=[pl.BlockSpec((BATCH // 8, HIST), lambda i: (i, 0))],
        out_specs=pl.BlockSpec((BATCH // 8, HIST), lambda i: (i, 0)),
    )(x)


def kernel(x, table):
    table_pad = _table_pad_tc(table)
    out_pad = _sc_embed(x, table_pad)
    out = out_pad[:, :, :EMB]
    mask = _mask_tc(x)
    return out, mask
